# Initial kernel scaffold; baseline (speedup 1.0000x reference)
import jax
import jax.numpy as jnp
from jax.experimental import pallas as pl


def kernel(x, tgt_pad, gate_w, cls_w, cls_b, w1, w2, w3):
    raise NotImplementedError("write your pallas kernel here")


# speedup vs baseline: 5.1705x; 5.1705x over previous
import jax
import jax.numpy as jnp
from jax.experimental import pallas as pl
from jax.experimental.pallas import tpu as pltpu

_B, _S, _DIM = 2, 2048, 1024
_E, _TOPK, _HID = 8, 2, 64
_EH = _E * _HID
_T = _B * _S
_TM = 512
_NBLK = _T // _TM


def _moe_block_kernel(x_ref, gate_wt_ref, w1c_ref, w3c_ref, w2c_ref,
                      cls_w_ref, cls_b_ref, out_ref, logits_ref):
    xb = x_ref[...]

    gl = jnp.dot(xb, gate_wt_ref[...], preferred_element_type=jnp.float32)
    gl = gl - jnp.max(gl, axis=-1, keepdims=True)
    p = jnp.exp(gl)
    p = p / jnp.sum(p, axis=-1, keepdims=True)
    eio = jax.lax.broadcasted_iota(jnp.int32, (_TM, _E), 1)
    m1 = jnp.max(p, axis=-1, keepdims=True)
    i1 = jnp.min(jnp.where(p >= m1, eio, _E), axis=-1, keepdims=True)
    pm = jnp.where(eio == i1, -1.0, p)
    m2 = jnp.max(pm, axis=-1, keepdims=True)
    i2 = jnp.min(jnp.where(pm >= m2, eio, _E), axis=-1, keepdims=True)
    denom = m1 + m2
    a1 = m1 / denom
    a2 = m2 / denom

    h1 = jnp.dot(xb, w1c_ref[...], preferred_element_type=jnp.float32)
    h3 = jnp.dot(xb, w3c_ref[...], preferred_element_type=jnp.float32)
    h = (h1 * jax.nn.sigmoid(h1)) * h3
    lane_e = jax.lax.broadcasted_iota(jnp.int32, (_TM, _EH), 1) // _HID
    w_exp = (jnp.where(lane_e == i1, a1, 0.0)
             + jnp.where(lane_e == i2, a2, 0.0))
    hw = h * w_exp
    out_ref[...] = jnp.dot(hw, w2c_ref[...],
                           preferred_element_type=jnp.float32)

    v = jnp.dot(w2c_ref[...], cls_w_ref[...],
                preferred_element_type=jnp.float32)
    lio = jax.lax.broadcasted_iota(jnp.int32, (_EH, _E), 0) // _HID
    ecol = jax.lax.broadcasted_iota(jnp.int32, (_EH, _E), 1)
    mcum = jnp.where(lio <= ecol, v, 0.0)
    lg = jnp.dot(hw[0:1, :], mcum,
                 preferred_element_type=jnp.float32) + cls_b_ref[...]
    logits_ref[...] = lg.reshape(1, 1, _E)


def kernel(x, tgt_pad, gate_w, cls_w, cls_b, w1, w2, w3):
    del tgt_pad
    xf = x.reshape(_T, _DIM)
    w1c = w1.reshape(_EH, _DIM).T
    w3c = w3.reshape(_EH, _DIM).T
    w2c = jnp.transpose(w2, (0, 2, 1)).reshape(_EH, _DIM)
    gate_wt = gate_w.T
    cls_wt = cls_w.T
    cls_b2 = cls_b.reshape(1, 1)

    out, logits = pl.pallas_call(
        _moe_block_kernel,
        grid=(_NBLK,),
        in_specs=[
            pl.BlockSpec((_TM, _DIM), lambda i: (i, 0)),
            pl.BlockSpec((_DIM, _E), lambda i: (0, 0)),
            pl.BlockSpec((_DIM, _EH), lambda i: (0, 0)),
            pl.BlockSpec((_DIM, _EH), lambda i: (0, 0)),
            pl.BlockSpec((_EH, _DIM), lambda i: (0, 0)),
            pl.BlockSpec((_DIM, 1), lambda i: (0, 0)),
            pl.BlockSpec((1, 1), lambda i: (0, 0)),
        ],
        out_specs=[
            pl.BlockSpec((_TM, _DIM), lambda i: (i, 0)),
            pl.BlockSpec((1, 1, _E), lambda i: (i, 0, 0)),
        ],
        out_shape=[
            jax.ShapeDtypeStruct((_T, _DIM), jnp.float32),
            jax.ShapeDtypeStruct((_NBLK, 1, _E), jnp.float32),
        ],
    )(xf, gate_wt, w1c, w3c, w2c, cls_wt, cls_b2)

    final_hidden_states = out.reshape(_B, _S, _DIM)
    el = logits[jnp.array([0, _S // _TM]), 0, :]
    expert_logits = el.T.reshape(_E, _B, 1)
    return final_hidden_states, expert_logits


# speedup vs baseline: 5.2210x; 1.0098x over previous
import jax
import jax.numpy as jnp
from jax.experimental import pallas as pl
from jax.experimental.pallas import tpu as pltpu

_B, _S, _DIM = 2, 2048, 1024
_E, _TOPK, _HID = 8, 2, 64
_EH = _E * _HID
_T = _B * _S
_TM = 512
_NBLK = _T // _TM


def _moe_block_kernel(x_ref, gate_wt_ref, w1c_ref, w3c_ref, w2c_ref,
                      cls_w_ref, cls_b_ref, out_ref, logits_ref):
    xb = x_ref[...]

    gl = jnp.dot(xb, gate_wt_ref[...], preferred_element_type=jnp.float32)
    gl = gl - jnp.max(gl, axis=-1, keepdims=True)
    p = jnp.exp(gl)
    p = p / jnp.sum(p, axis=-1, keepdims=True)
    eio = jax.lax.broadcasted_iota(jnp.int32, (_TM, _E), 1)
    m1 = jnp.max(p, axis=-1, keepdims=True)
    i1 = jnp.min(jnp.where(p >= m1, eio, _E), axis=-1, keepdims=True)
    pm = jnp.where(eio == i1, -1.0, p)
    m2 = jnp.max(pm, axis=-1, keepdims=True)
    i2 = jnp.min(jnp.where(pm >= m2, eio, _E), axis=-1, keepdims=True)
    denom = m1 + m2
    a1 = m1 / denom
    a2 = m2 / denom

    xb16 = xb.astype(jnp.bfloat16)
    h1 = jnp.dot(xb16, w1c_ref[...], preferred_element_type=jnp.float32)
    h3 = jnp.dot(xb16, w3c_ref[...], preferred_element_type=jnp.float32)
    h = (h1 * jax.nn.sigmoid(h1)) * h3
    lane_e = jax.lax.broadcasted_iota(jnp.int32, (_TM, _EH), 1) // _HID
    w_exp = (jnp.where(lane_e == i1, a1, 0.0)
             + jnp.where(lane_e == i2, a2, 0.0))
    hw = (h * w_exp).astype(jnp.bfloat16)
    out_ref[...] = jnp.dot(hw, w2c_ref[...],
                           preferred_element_type=jnp.float32)

    v = jnp.dot(w2c_ref[...], cls_w_ref[...].astype(jnp.bfloat16),
                preferred_element_type=jnp.float32)
    lio = jax.lax.broadcasted_iota(jnp.int32, (_EH, _E), 0) // _HID
    ecol = jax.lax.broadcasted_iota(jnp.int32, (_EH, _E), 1)
    mcum = jnp.where(lio <= ecol, v, 0.0)
    lg = jnp.dot(hw[0:1, :].astype(jnp.float32), mcum,
                 preferred_element_type=jnp.float32) + cls_b_ref[...]
    logits_ref[...] = lg.reshape(1, 1, _E)


def kernel(x, tgt_pad, gate_w, cls_w, cls_b, w1, w2, w3):
    del tgt_pad
    xf = x.reshape(_T, _DIM)
    w1c = w1.reshape(_EH, _DIM).T.astype(jnp.bfloat16)
    w3c = w3.reshape(_EH, _DIM).T.astype(jnp.bfloat16)
    w2c = jnp.transpose(w2, (0, 2, 1)).reshape(_EH, _DIM).astype(jnp.bfloat16)
    gate_wt = gate_w.T
    cls_wt = cls_w.T
    cls_b2 = cls_b.reshape(1, 1)

    out, logits = pl.pallas_call(
        _moe_block_kernel,
        grid=(_NBLK,),
        in_specs=[
            pl.BlockSpec((_TM, _DIM), lambda i: (i, 0)),
            pl.BlockSpec((_DIM, _E), lambda i: (0, 0)),
            pl.BlockSpec((_DIM, _EH), lambda i: (0, 0)),
            pl.BlockSpec((_DIM, _EH), lambda i: (0, 0)),
            pl.BlockSpec((_EH, _DIM), lambda i: (0, 0)),
            pl.BlockSpec((_DIM, 1), lambda i: (0, 0)),
            pl.BlockSpec((1, 1), lambda i: (0, 0)),
        ],
        out_specs=[
            pl.BlockSpec((_TM, _DIM), lambda i: (i, 0)),
            pl.BlockSpec((1, 1, _E), lambda i: (i, 0, 0)),
        ],
        out_shape=[
            jax.ShapeDtypeStruct((_T, _DIM), jnp.float32),
            jax.ShapeDtypeStruct((_NBLK, 1, _E), jnp.float32),
        ],
    )(xf, gate_wt, w1c, w3c, w2c, cls_wt, cls_b2)

    final_hidden_states = out.reshape(_B, _S, _DIM)
    el = logits[jnp.array([0, _S // _TM]), 0, :]
    expert_logits = el.T.reshape(_E, _B, 1)
    return final_hidden_states, expert_logits
